# baseline (speedup 1.0000x reference)
import functools

import jax
import jax.numpy as jnp
from jax.experimental import pallas as pl
from jax.experimental.pallas import tpu as pltpu

Q = 256
K = 512
BN = 10240
BN2 = 12800
CH = 2048
BIG_I = 2**30


def _scores_block(patch, block):
    b2row = jax.lax.dot_general(
        jnp.ones((1, K), jnp.float32), block * block,
        (((1,), (1,)), ((), ())),
        preferred_element_type=jnp.float32)
    prod = jax.lax.dot_general(
        -2.0 * patch, block, (((1,), (1,)), ((), ())),
        preferred_element_type=jnp.float32)
    return prod + b2row, b2row


def _pass1_body(nblocks, nrows, patch_ref, lib_ref, sstar_ref, cstar_ref,
                mtest_ref, b2_ref, minsq_ref, chk_ref):
    i = pl.program_id(0)
    patch = patch_ref[...]
    block = lib_ref[...]
    scores, b2row = _scores_block(patch, block)
    b2_ref[...] = b2row

    @pl.when(i == 0)
    def _():
        minsq_ref[...] = jnp.full((Q, 1), jnp.inf, jnp.float32)
        chk_ref[...] = jnp.zeros((Q, 1), jnp.int32)

    def update(sc):
        for j in range(BN // CH):
            bmin = jnp.min(sc[:, j * CH:(j + 1) * CH], axis=1).reshape(Q, 1)
            prev = minsq_ref[...]
            take = bmin < prev
            minsq_ref[...] = jnp.where(take, bmin, prev)
            chk_ref[...] = jnp.where(
                take, i * (BN // CH) + j, chk_ref[...])

    @pl.when(i < nblocks - 1)
    def _():
        update(scores)

    @pl.when(i == nblocks - 1)
    def _():
        cols = jax.lax.broadcasted_iota(jnp.int32, (1, BN), 1)
        valid = (cols + i * BN) < nrows
        update(jnp.where(valid, scores, jnp.float32(jnp.inf)))

        a2 = jnp.sum(patch * patch, axis=1).reshape(Q, 1)
        minval = jnp.sqrt(jnp.maximum(a2 + minsq_ref[...], 1e-12))
        sstar = jnp.max(minval)
        rows = jax.lax.broadcasted_iota(jnp.int32, (Q, 1), 0)
        s_idx = jnp.min(jnp.where(minval == sstar, rows, BIG_I))
        rowsel = rows == s_idx
        sstar_ref[...] = sstar.reshape(1, 1)
        cstar_ref[...] = jnp.sum(
            jnp.where(rowsel, chk_ref[...], 0)).reshape(1, 1)
        mtest_ref[...] = jnp.sum(
            jnp.where(rowsel, patch, 0.0), axis=0, keepdims=True)


def _pass2_body(nblocks, nrows, cs_ref, mtest_ref, sstar_ref, libA_ref,
                b2A_ref, lib_ref, b2_ref, out_ref, mstar_scr,
                vals_s, idx_s, tv_s):
    i = pl.program_id(0)
    mtest = mtest_ref[...]

    @pl.when(i == 0)
    def _():
        cstar = cs_ref[0]
        chunk = libA_ref[...]
        t0 = b2A_ref[...] + jax.lax.dot_general(
            -2.0 * mtest, chunk, (((1,), (1,)), ((), ())),
            preferred_element_type=jnp.float32)
        ccols = jax.lax.broadcasted_iota(jnp.int32, (1, CH), 1)
        t0 = jnp.where((ccols + cstar * CH) < nrows, t0,
                       jnp.float32(jnp.inf))
        m = jnp.min(t0)
        onehot = jnp.where(
            ccols == jnp.min(jnp.where(t0 == m, ccols, BIG_I)),
            1.0, 0.0)
        mstar_scr[...] = jax.lax.dot_general(
            onehot, chunk, (((1,), (0,)), ((), ())),
            preferred_element_type=jnp.float32)
        for k in range(3):
            vals_s[k] = jnp.float32(jnp.inf)
            idx_s[k] = jnp.int32(BIG_I + k)
            tv_s[k] = jnp.float32(0.0)

    @pl.when(i > 0)
    def _():
        ib = i - 1
        q = jnp.concatenate([mstar_scr[...], mtest], axis=0)
        block = lib_ref[...]
        b2row = b2_ref[...]
        prod = jax.lax.dot_general(
            -2.0 * q, block, (((1,), (1,)), ((), ())),
            preferred_element_type=jnp.float32)
        w = b2row + prod[0:1, :]
        a2_t = jnp.sum(mtest * mtest)
        t = a2_t + b2row + prod[1:2, :]

        cols = jax.lax.broadcasted_iota(jnp.int32, (1, BN2), 1)
        w = jnp.where((cols + ib * BN2) < nrows, w, jnp.float32(jnp.inf))
        cands = []
        for _ in range(3):
            m = jnp.min(w)
            loc = jnp.min(jnp.where(w == m, cols, BIG_I))
            sel = cols == loc
            tval = jnp.sum(jnp.where(sel, t, 0.0))
            cands.append((m, loc + ib * BN2, tval))
            w = jnp.where(sel, jnp.float32(jnp.inf), w)

        for k in range(3):
            cands.append((vals_s[k], idx_s[k], tv_s[k]))

        def cswap(a, b):
            sw = (b[0] < a[0]) | ((b[0] == a[0]) & (b[1] < a[1]))
            lo = tuple(jnp.where(sw, y, x) for x, y in zip(a, b))
            hi = tuple(jnp.where(sw, x, y) for x, y in zip(a, b))
            return lo, hi

        c = cands
        for p in range(6):
            for qi in range(5 - p):
                c[qi], c[qi + 1] = cswap(c[qi], c[qi + 1])

        for k in range(3):
            vals_s[k] = c[k][0]
            idx_s[k] = c[k][1]
            tv_s[k] = c[k][2]

    @pl.when(i == nblocks)
    def _():
        d = jnp.sqrt(jnp.float32(K))
        sstar = sstar_ref[0, 0]
        k1 = jnp.sqrt(jnp.maximum(tv_s[1], 0.0) + 1e-12)
        k2 = jnp.sqrt(jnp.maximum(tv_s[2], 0.0) + 1e-12)
        wgt = 1.0 - jnp.exp(sstar / d) / (jnp.exp(k1 / d) + jnp.exp(k2 / d))
        out_ref[...] = (wgt * sstar).reshape(1, 1)


@jax.jit
def kernel(patch, patch_lib):
    n = patch_lib.shape[0]
    nblocks = (n + BN - 1) // BN

    sstar, cstar, mtest, b2all = pl.pallas_call(
        functools.partial(_pass1_body, nblocks, n),
        grid=(nblocks,),
        in_specs=[
            pl.BlockSpec((Q, K), lambda i: (0, 0)),
            pl.BlockSpec((BN, K), lambda i: (i, 0)),
        ],
        out_specs=[
            pl.BlockSpec((1, 1), lambda i: (0, 0)),
            pl.BlockSpec((1, 1), lambda i: (0, 0)),
            pl.BlockSpec((1, K), lambda i: (0, 0)),
            pl.BlockSpec((1, BN), lambda i: (0, i)),
        ],
        out_shape=[
            jax.ShapeDtypeStruct((1, 1), jnp.float32),
            jax.ShapeDtypeStruct((1, 1), jnp.int32),
            jax.ShapeDtypeStruct((1, K), jnp.float32),
            jax.ShapeDtypeStruct((1, nblocks * BN), jnp.float32),
        ],
        scratch_shapes=[
            pltpu.VMEM((Q, 1), jnp.float32),
            pltpu.VMEM((Q, 1), jnp.int32),
        ],
    )(patch, patch_lib)

    nblocks2 = (n + BN2 - 1) // BN2
    s = pl.pallas_call(
        functools.partial(_pass2_body, nblocks2, n),
        grid_spec=pltpu.PrefetchScalarGridSpec(
            num_scalar_prefetch=1,
            grid=(nblocks2 + 1,),
            in_specs=[
                pl.BlockSpec((1, K), lambda i, cs: (0, 0)),
                pl.BlockSpec((1, 1), lambda i, cs: (0, 0)),
                pl.BlockSpec((CH, K), lambda i, cs: (cs[0], 0)),
                pl.BlockSpec((1, CH), lambda i, cs: (0, cs[0])),
                pl.BlockSpec((BN2, K),
                             lambda i, cs: (jnp.maximum(i - 1, 0), 0)),
                pl.BlockSpec((1, BN2),
                             lambda i, cs: (0, jnp.maximum(i - 1, 0))),
            ],
            out_specs=pl.BlockSpec((1, 1), lambda i, cs: (0, 0)),
            scratch_shapes=[
                pltpu.VMEM((1, K), jnp.float32),
                pltpu.SMEM((3,), jnp.float32),
                pltpu.SMEM((3,), jnp.int32),
                pltpu.SMEM((3,), jnp.float32),
            ],
        ),
        out_shape=jax.ShapeDtypeStruct((1, 1), jnp.float32),
    )(cstar.reshape(1), mtest, sstar, patch_lib, b2all, patch_lib, b2all)

    return s[0, 0]


# speedup vs baseline: 1.0065x; 1.0065x over previous
import functools

import jax
import jax.numpy as jnp
from jax.experimental import pallas as pl
from jax.experimental.pallas import tpu as pltpu

Q = 256
K = 512
BN = 10240
BN2 = 10240
CH = 2048
BIG_I = 2**30


def _scores_block(patch, block):
    b2row = jax.lax.dot_general(
        jnp.ones((1, K), jnp.float32), block * block,
        (((1,), (1,)), ((), ())),
        preferred_element_type=jnp.float32)
    prod = jax.lax.dot_general(
        -2.0 * patch, block, (((1,), (1,)), ((), ())),
        preferred_element_type=jnp.float32)
    return prod + b2row, b2row


def _pass1_body(nblocks, nrows, patch_ref, lib_ref, sstar_ref, cstar_ref,
                mtest_ref, b2_ref, minsq_ref, chk_ref):
    i = pl.program_id(0)
    patch = patch_ref[...]
    block = lib_ref[...]
    scores, b2row = _scores_block(patch, block)
    b2_ref[...] = b2row

    @pl.when(i == 0)
    def _():
        minsq_ref[...] = jnp.full((Q, 1), jnp.inf, jnp.float32)
        chk_ref[...] = jnp.zeros((Q, 1), jnp.int32)

    def update(sc):
        for j in range(BN // CH):
            bmin = jnp.min(sc[:, j * CH:(j + 1) * CH], axis=1).reshape(Q, 1)
            prev = minsq_ref[...]
            take = bmin < prev
            minsq_ref[...] = jnp.where(take, bmin, prev)
            chk_ref[...] = jnp.where(
                take, i * (BN // CH) + j, chk_ref[...])

    @pl.when(i < nblocks - 1)
    def _():
        update(scores)

    @pl.when(i == nblocks - 1)
    def _():
        cols = jax.lax.broadcasted_iota(jnp.int32, (1, BN), 1)
        valid = (cols + i * BN) < nrows
        update(jnp.where(valid, scores, jnp.float32(jnp.inf)))

        a2 = jnp.sum(patch * patch, axis=1).reshape(Q, 1)
        minval = jnp.sqrt(jnp.maximum(a2 + minsq_ref[...], 1e-12))
        sstar = jnp.max(minval)
        rows = jax.lax.broadcasted_iota(jnp.int32, (Q, 1), 0)
        s_idx = jnp.min(jnp.where(minval == sstar, rows, BIG_I))
        rowsel = rows == s_idx
        sstar_ref[...] = sstar.reshape(1, 1)
        cstar_ref[...] = jnp.sum(
            jnp.where(rowsel, chk_ref[...], 0)).reshape(1, 1)
        mtest_ref[...] = jnp.sum(
            jnp.where(rowsel, patch, 0.0), axis=0, keepdims=True)


def _pass2_body(nblocks, nrows, cs_ref, mtest_ref, sstar_ref, libA_ref,
                b2A_ref, lib_ref, b2_ref, out_ref, mstar_scr,
                vals_s, idx_s, tv_s):
    i = pl.program_id(0)
    mtest = mtest_ref[...]

    @pl.when(i == 0)
    def _():
        cstar = cs_ref[0]
        chunk = libA_ref[...]
        t0 = b2A_ref[...] + jax.lax.dot_general(
            -2.0 * mtest, chunk, (((1,), (1,)), ((), ())),
            preferred_element_type=jnp.float32)
        ccols = jax.lax.broadcasted_iota(jnp.int32, (1, CH), 1)
        t0 = jnp.where((ccols + cstar * CH) < nrows, t0,
                       jnp.float32(jnp.inf))
        m = jnp.min(t0)
        onehot = jnp.where(
            ccols == jnp.min(jnp.where(t0 == m, ccols, BIG_I)),
            1.0, 0.0)
        mstar_scr[...] = jax.lax.dot_general(
            onehot, chunk, (((1,), (0,)), ((), ())),
            preferred_element_type=jnp.float32)
        for k in range(3):
            vals_s[k] = jnp.float32(jnp.inf)
            idx_s[k] = jnp.int32(BIG_I + k)
            tv_s[k] = jnp.float32(0.0)

    @pl.when(i > 0)
    def _():
        ib = i - 1
        q = jnp.concatenate([mstar_scr[...], mtest], axis=0)
        block = lib_ref[...]
        b2row = b2_ref[...]
        prod = jax.lax.dot_general(
            -2.0 * q, block, (((1,), (1,)), ((), ())),
            preferred_element_type=jnp.float32)
        w = b2row + prod[0:1, :]
        a2_t = jnp.sum(mtest * mtest)
        t = a2_t + b2row + prod[1:2, :]

        cols = jax.lax.broadcasted_iota(jnp.int32, (1, BN2), 1)
        w = jnp.where((cols + ib * BN2) < nrows, w, jnp.float32(jnp.inf))
        cands = []
        for _ in range(3):
            m = jnp.min(w)
            loc = jnp.min(jnp.where(w == m, cols, BIG_I))
            sel = cols == loc
            tval = jnp.sum(jnp.where(sel, t, 0.0))
            cands.append((m, loc + ib * BN2, tval))
            w = jnp.where(sel, jnp.float32(jnp.inf), w)

        for k in range(3):
            cands.append((vals_s[k], idx_s[k], tv_s[k]))

        def cswap(a, b):
            sw = (b[0] < a[0]) | ((b[0] == a[0]) & (b[1] < a[1]))
            lo = tuple(jnp.where(sw, y, x) for x, y in zip(a, b))
            hi = tuple(jnp.where(sw, x, y) for x, y in zip(a, b))
            return lo, hi

        c = cands
        for p in range(6):
            for qi in range(5 - p):
                c[qi], c[qi + 1] = cswap(c[qi], c[qi + 1])

        for k in range(3):
            vals_s[k] = c[k][0]
            idx_s[k] = c[k][1]
            tv_s[k] = c[k][2]

    @pl.when(i == nblocks)
    def _():
        d = jnp.sqrt(jnp.float32(K))
        sstar = sstar_ref[0, 0]
        k1 = jnp.sqrt(jnp.maximum(tv_s[1], 0.0) + 1e-12)
        k2 = jnp.sqrt(jnp.maximum(tv_s[2], 0.0) + 1e-12)
        wgt = 1.0 - jnp.exp(sstar / d) / (jnp.exp(k1 / d) + jnp.exp(k2 / d))
        out_ref[...] = (wgt * sstar).reshape(1, 1)


@jax.jit
def kernel(patch, patch_lib):
    n = patch_lib.shape[0]
    nblocks = (n + BN - 1) // BN

    sstar, cstar, mtest, b2all = pl.pallas_call(
        functools.partial(_pass1_body, nblocks, n),
        grid=(nblocks,),
        in_specs=[
            pl.BlockSpec((Q, K), lambda i: (0, 0)),
            pl.BlockSpec((BN, K), lambda i: (i, 0)),
        ],
        out_specs=[
            pl.BlockSpec((1, 1), lambda i: (0, 0)),
            pl.BlockSpec((1, 1), lambda i: (0, 0)),
            pl.BlockSpec((1, K), lambda i: (0, 0)),
            pl.BlockSpec((1, BN), lambda i: (0, i)),
        ],
        out_shape=[
            jax.ShapeDtypeStruct((1, 1), jnp.float32),
            jax.ShapeDtypeStruct((1, 1), jnp.int32),
            jax.ShapeDtypeStruct((1, K), jnp.float32),
            jax.ShapeDtypeStruct((1, nblocks * BN), jnp.float32),
        ],
        scratch_shapes=[
            pltpu.VMEM((Q, 1), jnp.float32),
            pltpu.VMEM((Q, 1), jnp.int32),
        ],
    )(patch, patch_lib)

    nblocks2 = (n + BN2 - 1) // BN2
    s = pl.pallas_call(
        functools.partial(_pass2_body, nblocks2, n),
        grid_spec=pltpu.PrefetchScalarGridSpec(
            num_scalar_prefetch=1,
            grid=(nblocks2 + 1,),
            in_specs=[
                pl.BlockSpec((1, K), lambda i, cs: (0, 0)),
                pl.BlockSpec((1, 1), lambda i, cs: (0, 0)),
                pl.BlockSpec((CH, K), lambda i, cs: (cs[0], 0)),
                pl.BlockSpec((1, CH), lambda i, cs: (0, cs[0])),
                pl.BlockSpec((BN2, K),
                             lambda i, cs: (jnp.maximum(i - 1, 0), 0)),
                pl.BlockSpec((1, BN2),
                             lambda i, cs: (0, jnp.maximum(i - 1, 0))),
            ],
            out_specs=pl.BlockSpec((1, 1), lambda i, cs: (0, 0)),
            scratch_shapes=[
                pltpu.VMEM((1, K), jnp.float32),
                pltpu.SMEM((3,), jnp.float32),
                pltpu.SMEM((3,), jnp.int32),
                pltpu.SMEM((3,), jnp.float32),
            ],
        ),
        out_shape=jax.ShapeDtypeStruct((1, 1), jnp.float32),
    )(cstar.reshape(1), mtest, sstar, patch_lib, b2all, patch_lib, b2all)

    return s[0, 0]
